# Initial kernel scaffold; baseline (speedup 1.0000x reference)
import jax
import jax.numpy as jnp
from jax.experimental import pallas as pl


def kernel(inputs, table):
    raise NotImplementedError("write your pallas kernel here")


# speedup vs baseline: 1.0308x; 1.0308x over previous
import functools

import jax
import jax.numpy as jnp
from jax import lax
from jax.experimental import pallas as pl
from jax.experimental.pallas import tpu as pltpu
from jax.experimental.pallas import tpu_sc as plsc

_NC = 2
_NS = 16
_NW = _NC * _NS


@functools.lru_cache(maxsize=None)
def _make_gather(total, d, per_w, chunk, nchunks):
    mesh = plsc.VectorSubcoreMesh(core_axis_name="c", subcore_axis_name="s")

    @functools.partial(
        pl.kernel,
        mesh=mesh,
        out_type=jax.ShapeDtypeStruct((total, d), jnp.float32),
        scratch_types=[
            pltpu.VMEM((chunk,), jnp.int32),
            pltpu.VMEM((chunk, d), jnp.float32),
            pltpu.SemaphoreType.DMA,
        ],
        compiler_params=pltpu.CompilerParams(use_tc_tiling_on_sc=False),
    )
    def gather(ids_hbm, table_hbm, out_hbm, idx_v, rows_v, sem):
        wid = lax.axis_index("s") * _NC + lax.axis_index("c")
        base = wid * per_w

        def body(g, carry):
            off = base + g * chunk
            pltpu.sync_copy(ids_hbm.at[pl.ds(off, chunk)], idx_v)
            pltpu.async_copy(table_hbm.at[idx_v], rows_v, sem).wait()
            pltpu.sync_copy(rows_v, out_hbm.at[pl.ds(off, chunk)])
            return carry

        lax.fori_loop(0, nchunks, body, 0)

    return gather


def kernel(inputs, table):
    b, l = inputs.shape
    v, d = table.shape
    ids = jnp.concatenate(
        [jnp.full((b, 1), v - 1, inputs.dtype), inputs], axis=1)
    flat = ids.reshape(-1).astype(jnp.int32)
    total = b * (l + 1)
    per_w = total // _NW
    chunk = 1608
    nchunks = per_w // chunk
    out = _make_gather(total, d, per_w, chunk, nchunks)(flat, table)
    return out.reshape(b, l + 1, d)


# speedup vs baseline: 1.0310x; 1.0001x over previous
import functools

import jax
import jax.numpy as jnp
from jax import lax
from jax.experimental import pallas as pl
from jax.experimental.pallas import tpu as pltpu
from jax.experimental.pallas import tpu_sc as plsc

_NC = 2
_NS = 16
_NW = _NC * _NS
_NB = 2


@functools.lru_cache(maxsize=None)
def _make_gather(total, d, per_w, chunk, nchunks):
    mesh = plsc.VectorSubcoreMesh(core_axis_name="c", subcore_axis_name="s")
    n_outer = nchunks // _NB

    @functools.partial(
        pl.kernel,
        mesh=mesh,
        out_type=jax.ShapeDtypeStruct((total, d), jnp.float32),
        scratch_types=[
            pltpu.VMEM((nchunks, chunk), jnp.int32),
            pltpu.VMEM((_NB, chunk, d), jnp.float32),
            pltpu.SemaphoreType.DMA,
            pltpu.SemaphoreType.DMA,
            pltpu.SemaphoreType.DMA,
            pltpu.SemaphoreType.DMA,
        ],
        compiler_params=pltpu.CompilerParams(use_tc_tiling_on_sc=False),
    )
    def gather(ids_hbm, table_hbm, out_hbm, idx_all, rows_v, g0, g1, o0, o1):
        wid = lax.axis_index("s") * _NC + lax.axis_index("c")
        base = wid * per_w
        sem_g = [g0, g1]
        sem_o = [o0, o1]

        pltpu.sync_copy(ids_hbm.at[pl.ds(wid * nchunks, nchunks)], idx_all)

        def gather_copy(g, s):
            return pltpu.make_async_copy(
                table_hbm.at[idx_all.at[g]], rows_v.at[s], sem_g[s])

        def out_copy(g, s):
            return pltpu.make_async_copy(
                rows_v.at[s], out_hbm.at[pl.ds(base + g * chunk, chunk)],
                sem_o[s])

        def body(t, carry):
            for b in range(_NB):
                g = t * _NB + b
                @pl.when(t > 0)
                def _():
                    out_copy(g - _NB, b).wait()
                gather_copy(g, b).start()
                if b == 0:
                    @pl.when(t > 0)
                    def _():
                        gather_copy(g - 1, _NB - 1).wait()
                        out_copy(g - 1, _NB - 1).start()
                else:
                    gather_copy(g - 1, b - 1).wait()
                    out_copy(g - 1, b - 1).start()
            return carry

        lax.fori_loop(0, n_outer, body, 0)

        last = nchunks - 1
        gather_copy(last, _NB - 1).wait()
        out_copy(last, _NB - 1).start()
        out_copy(last - 1, 0).wait()
        out_copy(last, _NB - 1).wait()

    return gather


def kernel(inputs, table):
    b, l = inputs.shape
    v, d = table.shape
    ids = jnp.concatenate(
        [jnp.full((b, 1), v - 1, inputs.dtype), inputs], axis=1)
    total = b * (l + 1)
    per_w = total // _NW
    chunk = 1608
    nchunks = per_w // chunk
    flat = ids.reshape(total // chunk, chunk).astype(jnp.int32)
    out = _make_gather(total, d, per_w, chunk, nchunks)(flat, table)
    return out.reshape(b, l + 1, d)


# speedup vs baseline: 1.5608x; 1.5139x over previous
import functools

import jax
import jax.numpy as jnp
from jax import lax
from jax.experimental import pallas as pl
from jax.experimental.pallas import tpu as pltpu
from jax.experimental.pallas import tpu_sc as plsc

_NC = 2
_NS = 16
_NW = _NC * _NS
_BLK = 8


@functools.lru_cache(maxsize=None)
def _make_emb(b, l, v, d):
    per_w = b // _NW
    nblk = per_w // _BLK
    lp1 = l + 1
    mesh = plsc.VectorSubcoreMesh(core_axis_name="c", subcore_axis_name="s")

    @functools.partial(
        pl.kernel,
        mesh=mesh,
        out_type=jax.ShapeDtypeStruct((b, lp1, d), jnp.float32),
        scratch_types=[
            pltpu.VMEM((2, _BLK, l), jnp.int32),
            pltpu.VMEM((2, _BLK, lp1, d), jnp.float32),
            pltpu.VMEM((16,), jnp.int32),
            pltpu.VMEM((16, d), jnp.float32),
            pltpu.SemaphoreType.DMA,
            pltpu.SemaphoreType.DMA,
            pltpu.SemaphoreType.DMA,
            pltpu.SemaphoreType.DMA,
            pltpu.SemaphoreType.DMA,
        ],
        compiler_params=pltpu.CompilerParams(use_tc_tiling_on_sc=False),
    )
    def emb(in_hbm, table_hbm, out_hbm, idx_v, rows_v, cidx_v, crow_v,
            sem_c, g0, g1, o0, o1):
        wid = lax.axis_index("s") * _NC + lax.axis_index("c")
        base = wid * per_w
        sem_g = [g0, g1]
        sem_o = [o0, o1]

        cidx_v[...] = jnp.full((16,), v - 1, jnp.int32)
        pltpu.async_copy(table_hbm.at[cidx_v], crow_v, sem_c).wait()
        c0 = crow_v[0, pl.ds(0, 16)]
        c1 = crow_v[0, pl.ds(16, 16)]

        def gath(j, s):
            return pltpu.make_async_copy(
                table_hbm.at[idx_v.at[s, j]],
                rows_v.at[s, j, pl.ds(1, l)],
                sem_g[s])

        def out_copy(t, s):
            return pltpu.make_async_copy(
                rows_v.at[s],
                out_hbm.at[pl.ds(base + t * _BLK, _BLK)],
                sem_o[s])

        def body(tt, carry):
            for s in range(2):
                t = 2 * tt + s
                @pl.when(tt >= 1)
                def _():
                    out_copy(t - 2, s).wait()
                pltpu.sync_copy(in_hbm.at[pl.ds(base + t * _BLK, _BLK)],
                                idx_v.at[s])
                for j in range(_BLK):
                    rows_v[s, j, 0, pl.ds(0, 16)] = c0
                    rows_v[s, j, 0, pl.ds(16, 16)] = c1
                for j in range(_BLK):
                    gath(j, s).start()
                if s == 0:
                    @pl.when(tt >= 1)
                    def _():
                        for j in range(_BLK):
                            gath(j, 1).wait()
                        out_copy(t - 1, 1).start()
                else:
                    for j in range(_BLK):
                        gath(j, 0).wait()
                    out_copy(t - 1, 0).start()
            return carry

        lax.fori_loop(0, nblk // 2, body, 0)

        for j in range(_BLK):
            gath(j, 1).wait()
        out_copy(nblk - 1, 1).start()
        out_copy(nblk - 2, 0).wait()
        out_copy(nblk - 1, 1).wait()

    return emb


def kernel(inputs, table):
    b, l = inputs.shape
    v, d = table.shape
    return _make_emb(b, l, v, d)(inputs.astype(jnp.int32), table)
